# baseline (speedup 1.0000x reference)
import functools

import jax
import jax.numpy as jnp
from jax import lax
from jax.experimental import pallas as pl
from jax.experimental.pallas import tpu as pltpu
from jax.experimental.pallas import tpu_sc as plsc

_B, _S, _D = 4, 4096, 1024
_NC, _NS, _L = 2, 16, 16
_NW = _NC * _NS
_RPW = (_B * _S) // _NW
_CH = 32
_NCHUNK = _RPW // _CH


def _sc_body(x_hbm, pos_hbm, out_hbm, x_v, pos_v):
    wid = lax.axis_index("s") * _NC + lax.axis_index("c")
    base = wid * _RPW

    def chunk(k, carry):
        row0 = base + k * _CH
        prow0 = lax.rem(row0, _S)
        pltpu.sync_copy(x_hbm.at[pl.ds(row0, _CH)], x_v)
        pltpu.sync_copy(pos_hbm.at[pl.ds(prow0, _CH)], pos_v)

        def add_row(r, c2):
            for c in range(_D // _L):
                sl = pl.ds(c * _L, _L)
                x_v[r, sl] = x_v[r, sl] + pos_v[r, sl]
            return c2

        lax.fori_loop(0, _CH, add_row, 0)
        pltpu.sync_copy(x_v, out_hbm.at[pl.ds(row0, _CH)])
        return carry

    lax.fori_loop(0, _NCHUNK, chunk, 0)


def kernel(x, pos_table):
    B, S, D = x.shape
    x2d = x.reshape(B * S, D)
    mesh = plsc.VectorSubcoreMesh(core_axis_name="c", subcore_axis_name="s")
    run = functools.partial(
        pl.kernel,
        mesh=mesh,
        out_type=jax.ShapeDtypeStruct((B * S, D), x.dtype),
        scratch_types=[
            pltpu.VMEM((_CH, D), jnp.float32),
            pltpu.VMEM((_CH, D), jnp.float32),
        ],
    )(_sc_body)
    out2d = run(x2d, pos_table)
    return out2d.reshape(B, S, D)


# speedup vs baseline: 1.4937x; 1.4937x over previous
import functools

import jax
import jax.numpy as jnp
from jax import lax
from jax.experimental import pallas as pl
from jax.experimental.pallas import tpu as pltpu
from jax.experimental.pallas import tpu_sc as plsc

_B, _S, _D = 4, 4096, 1024
_NC, _NS, _L = 2, 16, 16
_NW = _NC * _NS
_SPW = _S // _NW
_CH = 16
_NSC = _SPW // _CH
_NT = _NSC * _B


def _sc_body(x_hbm, pos_hbm, out_hbm,
             xb0, xb1, pb0, pb1, sx0, sx1, sp0, sp1, so0, so1):
    wid = lax.axis_index("s") * _NC + lax.axis_index("c")
    sbase = wid * _SPW
    xb = (xb0, xb1)
    pb = (pb0, pb1)
    sx = (sx0, sx1)
    sp = (sp0, sp1)
    so = (so0, so1)

    def xrow0(t):
        sc, b = divmod(t, _B)
        return b * _S + sbase + sc * _CH

    dx = [None] * _NT
    dp = [None] * _NSC
    do = [None] * _NT

    def start_in(t):
        i = t % 2
        d = pltpu.make_async_copy(x_hbm.at[pl.ds(xrow0(t), _CH)], xb[i], sx[i])
        d.start()
        dx[t] = d
        sc, b = divmod(t, _B)
        if b == 0:
            j = sc % 2
            d = pltpu.make_async_copy(
                pos_hbm.at[pl.ds(sbase + sc * _CH, _CH)], pb[j], sp[j])
            d.start()
            dp[sc] = d

    start_in(0)
    for t in range(_NT):
        i = t % 2
        if t + 1 < _NT:
            if t >= 1:
                do[t - 1].wait()
            start_in(t + 1)
        dx[t].wait()
        sc, b = divmod(t, _B)
        if b == 0:
            dp[sc].wait()
        pbuf = pb[sc % 2]
        xbuf = xb[i]

        def add_row(r, carry):
            for c in range(_D // _L):
                sl = pl.ds(c * _L, _L)
                xbuf[r, sl] = xbuf[r, sl] + pbuf[r, sl]
            return carry

        lax.fori_loop(0, _CH, add_row, 0)
        d = pltpu.make_async_copy(xbuf, out_hbm.at[pl.ds(xrow0(t), _CH)], so[i])
        d.start()
        do[t] = d
    do[_NT - 2].wait()
    do[_NT - 1].wait()


def kernel(x, pos_table):
    B, S, D = x.shape
    x2d = x.reshape(B * S, D)
    mesh = plsc.VectorSubcoreMesh(core_axis_name="c", subcore_axis_name="s")
    run = functools.partial(
        pl.kernel,
        mesh=mesh,
        out_type=jax.ShapeDtypeStruct((B * S, D), x.dtype),
        scratch_types=[
            pltpu.VMEM((_CH, D), jnp.float32),
            pltpu.VMEM((_CH, D), jnp.float32),
            pltpu.VMEM((_CH, D), jnp.float32),
            pltpu.VMEM((_CH, D), jnp.float32),
            pltpu.SemaphoreType.DMA,
            pltpu.SemaphoreType.DMA,
            pltpu.SemaphoreType.DMA,
            pltpu.SemaphoreType.DMA,
            pltpu.SemaphoreType.DMA,
            pltpu.SemaphoreType.DMA,
        ],
    )(_sc_body)
    out2d = run(x2d, pos_table)
    return out2d.reshape(B, S, D)


# speedup vs baseline: 3.3323x; 2.2309x over previous
import jax
import jax.numpy as jnp
from jax.experimental import pallas as pl
from jax.experimental.pallas import tpu as pltpu

_B, _S, _D = 4, 4096, 1024
_CH = 512
_NSC = _S // _CH
_NT = _NSC * _B
_NBUF = 4


def _pipe_kernel(x_hbm, pos_hbm, o_hbm, xb, pb,
                 sx0, sx1, sx2, sx3, sp0, sp1, so0, so1, so2, so3):
    sx = (sx0, sx1, sx2, sx3)
    sp = (sp0, sp1)
    so = (so0, so1, so2, so3)

    def xrow0(t):
        sc, b = divmod(t, _B)
        return b * _S + sc * _CH

    dx = [None] * _NT
    dp = [None] * _NSC
    do = [None] * _NT

    def start_in(t):
        i = t % _NBUF
        d = pltpu.make_async_copy(
            x_hbm.at[pl.ds(xrow0(t), _CH)], xb.at[i], sx[i])
        d.start()
        dx[t] = d
        sc, b = divmod(t, _B)
        if b == 0:
            j = sc % 2
            d = pltpu.make_async_copy(
                pos_hbm.at[pl.ds(sc * _CH, _CH)], pb.at[j], sp[j])
            d.start()
            dp[sc] = d

    start_in(0)
    start_in(1)
    for t in range(_NT):
        i = t % _NBUF
        if t + 2 < _NT:
            if t - 2 >= 0:
                do[t - 2].wait()
            start_in(t + 2)
        dx[t].wait()
        sc, b = divmod(t, _B)
        if b == 0:
            dp[sc].wait()
        xv = xb.at[i]
        xv[...] = xv[...] + pb[sc % 2]
        d = pltpu.make_async_copy(xv, o_hbm.at[pl.ds(xrow0(t), _CH)], so[i])
        d.start()
        do[t] = d
    for t in range(_NT - 4, _NT):
        do[t].wait()


def kernel(x, pos_table):
    B, S, D = x.shape
    x2d = x.reshape(B * S, D)
    out2d = pl.pallas_call(
        _pipe_kernel,
        in_specs=[
            pl.BlockSpec(memory_space=pltpu.MemorySpace.HBM),
            pl.BlockSpec(memory_space=pltpu.MemorySpace.HBM),
        ],
        out_specs=pl.BlockSpec(memory_space=pltpu.MemorySpace.HBM),
        out_shape=jax.ShapeDtypeStruct((B * S, D), x.dtype),
        scratch_shapes=[
            pltpu.VMEM((_NBUF, _CH, D), jnp.float32),
            pltpu.VMEM((2, _CH, D), jnp.float32),
            pltpu.SemaphoreType.DMA,
            pltpu.SemaphoreType.DMA,
            pltpu.SemaphoreType.DMA,
            pltpu.SemaphoreType.DMA,
            pltpu.SemaphoreType.DMA,
            pltpu.SemaphoreType.DMA,
            pltpu.SemaphoreType.DMA,
            pltpu.SemaphoreType.DMA,
            pltpu.SemaphoreType.DMA,
            pltpu.SemaphoreType.DMA,
        ],
    )(x2d, pos_table)
    return out2d.reshape(B, S, D)


# speedup vs baseline: 3.4418x; 1.0329x over previous
import jax
import jax.numpy as jnp
from jax.experimental import pallas as pl
from jax.experimental.pallas import tpu as pltpu

_B, _S, _D = 4, 4096, 1024
_CH = 1024
_NSC = _S // _CH
_NT = _NSC * _B
_NBUF = 4


def _pipe_kernel(x_hbm, pos_hbm, o_hbm, xb, pb,
                 sx0, sx1, sx2, sx3, sp0, sp1, so0, so1, so2, so3):
    sx = (sx0, sx1, sx2, sx3)
    sp = (sp0, sp1)
    so = (so0, so1, so2, so3)

    def xrow0(t):
        sc, b = divmod(t, _B)
        return b * _S + sc * _CH

    dx = [None] * _NT
    dp = [None] * _NSC
    do = [None] * _NT

    def start_in(t):
        i = t % _NBUF
        d = pltpu.make_async_copy(
            x_hbm.at[pl.ds(xrow0(t), _CH)], xb.at[i], sx[i])
        d.start()
        dx[t] = d
        sc, b = divmod(t, _B)
        if b == 0:
            j = sc % 2
            d = pltpu.make_async_copy(
                pos_hbm.at[pl.ds(sc * _CH, _CH)], pb.at[j], sp[j])
            d.start()
            dp[sc] = d

    start_in(0)
    start_in(1)
    for t in range(_NT):
        i = t % _NBUF
        if t + 2 < _NT:
            if t - 2 >= 0:
                do[t - 2].wait()
            start_in(t + 2)
        dx[t].wait()
        sc, b = divmod(t, _B)
        if b == 0:
            dp[sc].wait()
        xv = xb.at[i]
        xv[...] = xv[...] + pb[sc % 2]
        d = pltpu.make_async_copy(xv, o_hbm.at[pl.ds(xrow0(t), _CH)], so[i])
        d.start()
        do[t] = d
    for t in range(_NT - 4, _NT):
        do[t].wait()


def kernel(x, pos_table):
    B, S, D = x.shape
    x2d = x.reshape(B * S, D)
    out2d = pl.pallas_call(
        _pipe_kernel,
        in_specs=[
            pl.BlockSpec(memory_space=pltpu.MemorySpace.HBM),
            pl.BlockSpec(memory_space=pltpu.MemorySpace.HBM),
        ],
        out_specs=pl.BlockSpec(memory_space=pltpu.MemorySpace.HBM),
        out_shape=jax.ShapeDtypeStruct((B * S, D), x.dtype),
        scratch_shapes=[
            pltpu.VMEM((_NBUF, _CH, D), jnp.float32),
            pltpu.VMEM((2, _CH, D), jnp.float32),
            pltpu.SemaphoreType.DMA,
            pltpu.SemaphoreType.DMA,
            pltpu.SemaphoreType.DMA,
            pltpu.SemaphoreType.DMA,
            pltpu.SemaphoreType.DMA,
            pltpu.SemaphoreType.DMA,
            pltpu.SemaphoreType.DMA,
            pltpu.SemaphoreType.DMA,
            pltpu.SemaphoreType.DMA,
            pltpu.SemaphoreType.DMA,
        ],
    )(x2d, pos_table)
    return out2d.reshape(B, S, D)


# speedup vs baseline: 3.5102x; 1.0199x over previous
import jax
import jax.numpy as jnp
from jax.experimental import pallas as pl
from jax.experimental.pallas import tpu as pltpu

_B, _S, _D = 4, 4096, 1024
_CH = 2048
_NSC = _S // _CH
_NT = _NSC * _B
_NBUF = 4


def _pipe_kernel(x_hbm, pos_hbm, o_hbm, xb, pb,
                 sx0, sx1, sx2, sx3, sp0, sp1, so0, so1, so2, so3):
    sx = (sx0, sx1, sx2, sx3)
    sp = (sp0, sp1)
    so = (so0, so1, so2, so3)

    def xrow0(t):
        sc, b = divmod(t, _B)
        return b * _S + sc * _CH

    dx = [None] * _NT
    dp = [None] * _NSC
    do = [None] * _NT

    def start_in(t):
        i = t % _NBUF
        d = pltpu.make_async_copy(
            x_hbm.at[pl.ds(xrow0(t), _CH)], xb.at[i], sx[i])
        d.start()
        dx[t] = d
        sc, b = divmod(t, _B)
        if b == 0:
            j = sc % 2
            d = pltpu.make_async_copy(
                pos_hbm.at[pl.ds(sc * _CH, _CH)], pb.at[j], sp[j])
            d.start()
            dp[sc] = d

    start_in(0)
    start_in(1)
    for t in range(_NT):
        i = t % _NBUF
        if t + 2 < _NT:
            if t - 2 >= 0:
                do[t - 2].wait()
            start_in(t + 2)
        dx[t].wait()
        sc, b = divmod(t, _B)
        if b == 0:
            dp[sc].wait()
        xv = xb.at[i]
        xv[...] = xv[...] + pb[sc % 2]
        d = pltpu.make_async_copy(xv, o_hbm.at[pl.ds(xrow0(t), _CH)], so[i])
        d.start()
        do[t] = d
    for t in range(_NT - 4, _NT):
        do[t].wait()


def kernel(x, pos_table):
    B, S, D = x.shape
    x2d = x.reshape(B * S, D)
    out2d = pl.pallas_call(
        _pipe_kernel,
        in_specs=[
            pl.BlockSpec(memory_space=pltpu.MemorySpace.HBM),
            pl.BlockSpec(memory_space=pltpu.MemorySpace.HBM),
        ],
        out_specs=pl.BlockSpec(memory_space=pltpu.MemorySpace.HBM),
        out_shape=jax.ShapeDtypeStruct((B * S, D), x.dtype),
        scratch_shapes=[
            pltpu.VMEM((_NBUF, _CH, D), jnp.float32),
            pltpu.VMEM((2, _CH, D), jnp.float32),
            pltpu.SemaphoreType.DMA,
            pltpu.SemaphoreType.DMA,
            pltpu.SemaphoreType.DMA,
            pltpu.SemaphoreType.DMA,
            pltpu.SemaphoreType.DMA,
            pltpu.SemaphoreType.DMA,
            pltpu.SemaphoreType.DMA,
            pltpu.SemaphoreType.DMA,
            pltpu.SemaphoreType.DMA,
            pltpu.SemaphoreType.DMA,
        ],
    )(x2d, pos_table)
    return out2d.reshape(B, S, D)
